# Initial kernel scaffold; baseline (speedup 1.0000x reference)
import jax
import jax.numpy as jnp
from jax.experimental import pallas as pl


def kernel(batch, charge, temperature, emb_charge, W1, W2, Wp):
    raise NotImplementedError("write your pallas kernel here")


# speedup vs baseline: 3.8440x; 3.8440x over previous
import functools

import jax
import jax.numpy as jnp
from jax import lax
from jax.experimental import pallas as pl
from jax.experimental.pallas import tpu as pltpu
from jax.experimental.pallas import tpu_sc as plsc

_NC = 2
_NS = 16
_NW = _NC * _NS
_G = 128


def _table_body(charge_ref, temp_ref, emb_ref, w1_ref, w2_ref, wpa_ref,
                wpb_ref, out_ref):
    b = charge_ref.shape[0]
    vp = emb_ref.shape[0]
    charge = charge_ref[...]
    iota = lax.broadcasted_iota(jnp.int32, (b, vp), 1)
    oh = jnp.where(charge == iota, 1.0, 0.0).astype(jnp.float32)
    e_charge = jnp.dot(oh, emb_ref[...],
                       preferred_element_type=jnp.float32)
    t = temp_ref[...]
    h = t * w1_ref[...]
    h = h * jax.nn.sigmoid(h)
    e_temp = jnp.dot(h, w2_ref[...],
                     preferred_element_type=jnp.float32)
    y = (jnp.dot(e_charge, wpa_ref[...], preferred_element_type=jnp.float32)
         + jnp.dot(e_temp, wpb_ref[...], preferred_element_type=jnp.float32))
    out_ref[...] = y * jax.nn.sigmoid(y)


def _make_gather(n_pad, d):
    n_g = n_pad // (_NW * _G)
    rows_per_w = n_g * _G
    mesh = plsc.VectorSubcoreMesh(core_axis_name="c", subcore_axis_name="s")

    @functools.partial(
        pl.kernel,
        mesh=mesh,
        out_type=jax.ShapeDtypeStruct((n_pad, d), jnp.float32),
        scratch_types=[
            pltpu.VMEM((n_g, _G), jnp.int32),
            pltpu.VMEM((_G, d), jnp.float32),
            pltpu.SemaphoreType.DMA,
        ],
        compiler_params=pltpu.CompilerParams(use_tc_tiling_on_sc=False),
    )
    def gather_k(table_hbm, idx_hbm, out_hbm, idx_v, rows_v, sem):
        wid = lax.axis_index("s") * _NC + lax.axis_index("c")
        pltpu.sync_copy(idx_hbm.at[wid], idx_v)
        base = wid * rows_per_w

        def body(j, carry):
            pltpu.async_copy(table_hbm.at[idx_v.at[j]], rows_v, sem).wait()
            pltpu.sync_copy(rows_v, out_hbm.at[pl.ds(base + j * _G, _G)])
            return carry

        lax.fori_loop(0, n_g, body, 0, unroll=False)

    return gather_k


def kernel(batch, charge, temperature, emb_charge, W1, W2, Wp):
    n = batch.shape[0]
    b = charge.shape[0]
    v, d = emb_charge.shape

    vp = ((v + 127) // 128) * 128
    emb_pad = jnp.zeros((vp, d), jnp.float32).at[:v].set(emb_charge)
    table = pl.pallas_call(
        _table_body,
        out_shape=jax.ShapeDtypeStruct((b, d), jnp.float32),
    )(charge.astype(jnp.int32).reshape(b, 1),
      temperature.reshape(b, 1),
      emb_pad, W1, W2, Wp[:d, :], Wp[d:, :])

    chunk = _NW * _G
    n_pad = ((n + chunk - 1) // chunk) * chunk
    idx = jnp.zeros((n_pad,), jnp.int32).at[:n].set(batch.astype(jnp.int32))
    n_g = n_pad // chunk
    idx3 = idx.reshape(_NW, n_g, _G)
    out = _make_gather(n_pad, d)(table, idx3)
    return out[:n]


# speedup vs baseline: 5.1535x; 1.3407x over previous
import functools

import jax
import jax.numpy as jnp
from jax import lax
from jax.experimental import pallas as pl
from jax.experimental.pallas import tpu as pltpu
from jax.experimental.pallas import tpu_sc as plsc

_NC = 2
_NS = 16
_NW = _NC * _NS
_G = 112
_NBUF = 4


def _table_body(charge_ref, temp_ref, emb_ref, w1_ref, w2_ref, wpa_ref,
                wpb_ref, out_ref):
    b = charge_ref.shape[0]
    vp = emb_ref.shape[0]
    charge = charge_ref[...]
    iota = lax.broadcasted_iota(jnp.int32, (b, vp), 1)
    oh = jnp.where(charge == iota, 1.0, 0.0).astype(jnp.float32)
    e_charge = jnp.dot(oh, emb_ref[...],
                       preferred_element_type=jnp.float32)
    t = temp_ref[...]
    h = t * w1_ref[...]
    h = h * jax.nn.sigmoid(h)
    e_temp = jnp.dot(h, w2_ref[...],
                     preferred_element_type=jnp.float32)
    y = (jnp.dot(e_charge, wpa_ref[...], preferred_element_type=jnp.float32)
         + jnp.dot(e_temp, wpb_ref[...], preferred_element_type=jnp.float32))
    out_ref[...] = y * jax.nn.sigmoid(y)


def _make_gather(n, d, n_g):
    rows_per_w = n_g * _G
    mesh = plsc.VectorSubcoreMesh(core_axis_name="c", subcore_axis_name="s")

    @functools.partial(
        pl.kernel,
        mesh=mesh,
        out_type=jax.ShapeDtypeStruct((n, d), jnp.float32),
        scratch_types=[
            pltpu.VMEM((n_g, _G), jnp.int32),
        ] + [pltpu.VMEM((_G, d), jnp.float32) for _ in range(_NBUF)]
          + [pltpu.SemaphoreType.DMA for _ in range(2 * _NBUF)],
        compiler_params=pltpu.CompilerParams(use_tc_tiling_on_sc=False),
    )
    def gather_k(table_hbm, idx_hbm, out_hbm, idx_v,
                 r0, r1, r2, r3, g0, g1, g2, g3, s0, s1, s2, s3):
        rows = (r0, r1, r2, r3)
        gs = (g0, g1, g2, g3)
        ss = (s0, s1, s2, s3)
        wid = lax.axis_index("s") * _NC + lax.axis_index("c")
        pltpu.sync_copy(idx_hbm.at[wid], idx_v)
        base = wid * rows_per_w

        def off(j):
            return jnp.minimum(base + j * _G, n - _G)

        def fire_gather(j, p):
            pltpu.async_copy(table_hbm.at[idx_v.at[j]], rows[p], gs[p])

        def wait_gather(j, p):
            pltpu.make_async_copy(table_hbm.at[idx_v.at[j]], rows[p],
                                  gs[p]).wait()

        def fire_store(j, p):
            pltpu.async_copy(rows[p], out_hbm.at[pl.ds(off(j), _G)], ss[p])

        def wait_store(j, p):
            pltpu.make_async_copy(rows[p], out_hbm.at[pl.ds(off(j), _G)],
                                  ss[p]).wait()

        fire_gather(0, 0)
        fire_gather(1, 1)

        def quad(jj, carry):
            for bq in range(_NBUF):
                j = jj * _NBUF + bq
                p = bq
                q = (bq + 2) % _NBUF

                @pl.when(j >= 2)
                def _():
                    wait_store(j - 2, q)

                @pl.when(j + 2 < n_g)
                def _():
                    fire_gather(j + 2, q)

                wait_gather(j, p)
                fire_store(j, p)
            return carry

        lax.fori_loop(0, n_g // _NBUF, quad, 0, unroll=False)
        wait_store(n_g - 2, (n_g - 2) % _NBUF)
        wait_store(n_g - 1, (n_g - 1) % _NBUF)

    return gather_k


def kernel(batch, charge, temperature, emb_charge, W1, W2, Wp):
    n = batch.shape[0]
    b = charge.shape[0]
    v, d = emb_charge.shape

    vp = ((v + 127) // 128) * 128
    emb_pad = jnp.zeros((vp, d), jnp.float32).at[:v].set(emb_charge)
    table = pl.pallas_call(
        _table_body,
        out_shape=jax.ShapeDtypeStruct((b, d), jnp.float32),
    )(charge.astype(jnp.int32).reshape(b, 1),
      temperature.reshape(b, 1),
      emb_pad, W1, W2, Wp[:d, :], Wp[d:, :])

    chunk = _NW * _G
    n_g = -(-n // chunk)
    n_g = ((n_g + _NBUF - 1) // _NBUF) * _NBUF
    batch_i32 = batch.astype(jnp.int32)
    w_ids = jnp.arange(_NW, dtype=jnp.int32)[:, None]
    j_ids = jnp.arange(n_g, dtype=jnp.int32)[None, :]
    offs = jnp.minimum(w_ids * (n_g * _G) + j_ids * _G, n - _G)
    pos = offs[:, :, None] + jnp.arange(_G, dtype=jnp.int32)
    idx3 = batch_i32[pos]
    return _make_gather(n, d, n_g)(table, idx3)


# speedup vs baseline: 5.8518x; 1.1355x over previous
import functools

import jax
import jax.numpy as jnp
from jax import lax
from jax.experimental import pallas as pl
from jax.experimental.pallas import tpu as pltpu
from jax.experimental.pallas import tpu_sc as plsc

_NC = 2
_NS = 16
_NW = _NC * _NS
_G = 112
_NBUF = 7
_FA = 3


def _table_body(charge_ref, temp_ref, emb_ref, w1_ref, w2_ref, wpa_ref,
                wpb_ref, out_ref):
    b = charge_ref.shape[0]
    vp = emb_ref.shape[0]
    charge = charge_ref[...]
    iota = lax.broadcasted_iota(jnp.int32, (b, vp), 1)
    oh = jnp.where(charge == iota, 1.0, 0.0).astype(jnp.float32)
    e_charge = jnp.dot(oh, emb_ref[...],
                       preferred_element_type=jnp.float32)
    t = temp_ref[...]
    h = t * w1_ref[...]
    h = h * jax.nn.sigmoid(h)
    e_temp = jnp.dot(h, w2_ref[...],
                     preferred_element_type=jnp.float32)
    y = (jnp.dot(e_charge, wpa_ref[...], preferred_element_type=jnp.float32)
         + jnp.dot(e_temp, wpb_ref[...], preferred_element_type=jnp.float32))
    out_ref[...] = y * jax.nn.sigmoid(y)


def _make_gather(n, d, n_g):
    rows_per_w = n_g * _G
    mesh = plsc.VectorSubcoreMesh(core_axis_name="c", subcore_axis_name="s")

    @functools.partial(
        pl.kernel,
        mesh=mesh,
        out_type=jax.ShapeDtypeStruct((n, d), jnp.float32),
        scratch_types=[
            pltpu.VMEM((rows_per_w,), jnp.int32),
        ] + [pltpu.VMEM((_G, d), jnp.float32) for _ in range(_NBUF)]
          + [pltpu.SemaphoreType.DMA for _ in range(2 * _NBUF)],
        compiler_params=pltpu.CompilerParams(use_tc_tiling_on_sc=False),
    )
    def gather_k(table_hbm, batch_hbm, out_hbm, idx_v, *bufs):
        rows = bufs[:_NBUF]
        gs = bufs[_NBUF:2 * _NBUF]
        ss = bufs[2 * _NBUF:]
        wid = lax.axis_index("s") * _NC + lax.axis_index("c")
        woff = jnp.minimum(wid * rows_per_w, n - rows_per_w)
        pltpu.sync_copy(batch_hbm.at[pl.ds(woff, rows_per_w)], idx_v)

        def fire_gather(j, p):
            pltpu.async_copy(table_hbm.at[idx_v.at[pl.ds(j * _G, _G)]],
                             rows[p], gs[p])

        def wait_gather(j, p):
            pltpu.make_async_copy(table_hbm.at[idx_v.at[pl.ds(j * _G, _G)]],
                                  rows[p], gs[p]).wait()

        def fire_store(j, p):
            pltpu.async_copy(rows[p], out_hbm.at[pl.ds(woff + j * _G, _G)],
                             ss[p])

        def wait_store(j, p):
            pltpu.make_async_copy(rows[p],
                                  out_hbm.at[pl.ds(woff + j * _G, _G)],
                                  ss[p]).wait()

        for j in range(_FA):
            fire_gather(j, j)

        def ring(jj, carry):
            for bq in range(_NBUF):
                j = jj * _NBUF + bq
                q = (bq + _FA) % _NBUF

                @pl.when(j >= _NBUF - _FA)
                def _():
                    wait_store(j - (_NBUF - _FA), q)

                @pl.when(j + _FA < n_g)
                def _():
                    fire_gather(j + _FA, q)

                wait_gather(j, bq)
                fire_store(j, bq)
            return carry

        lax.fori_loop(0, n_g // _NBUF, ring, 0, unroll=False)
        for j in range(n_g - (_NBUF - _FA), n_g):
            wait_store(j, j % _NBUF)

    return gather_k


def kernel(batch, charge, temperature, emb_charge, W1, W2, Wp):
    n = batch.shape[0]
    b = charge.shape[0]
    v, d = emb_charge.shape

    vp = ((v + 127) // 128) * 128
    emb_pad = jnp.zeros((vp, d), jnp.float32).at[:v].set(emb_charge)
    table = pl.pallas_call(
        _table_body,
        out_shape=jax.ShapeDtypeStruct((b, d), jnp.float32),
    )(charge.astype(jnp.int32).reshape(b, 1),
      temperature.reshape(b, 1),
      emb_pad, W1, W2, Wp[:d, :], Wp[d:, :])

    chunk = _NW * _G
    n_g = -(-n // chunk)
    n_g = ((n_g + _NBUF - 1) // _NBUF) * _NBUF
    assert n_g * _G <= n, "worker window exceeds output"
    return _make_gather(n, d, n_g)(table, batch.astype(jnp.int32))


# speedup vs baseline: 6.0183x; 1.0285x over previous
import functools

import jax
import jax.numpy as jnp
from jax import lax
from jax.experimental import pallas as pl
from jax.experimental.pallas import tpu as pltpu
from jax.experimental.pallas import tpu_sc as plsc

_NC = 2
_NS = 16
_NW = _NC * _NS
_G = 112
_NBUF = 7
_FA = 4


def _table_body(charge_ref, temp_ref, emb_ref, w1_ref, w2_ref, wpa_ref,
                wpb_ref, out_ref):
    b = charge_ref.shape[0]
    vp = emb_ref.shape[0]
    charge = charge_ref[...]
    iota = lax.broadcasted_iota(jnp.int32, (b, vp), 1)
    oh = jnp.where(charge == iota, 1.0, 0.0).astype(jnp.float32)
    e_charge = jnp.dot(oh, emb_ref[...],
                       preferred_element_type=jnp.float32)
    t = temp_ref[...]
    h = t * w1_ref[...]
    h = h * jax.nn.sigmoid(h)
    e_temp = jnp.dot(h, w2_ref[...],
                     preferred_element_type=jnp.float32)
    y = (jnp.dot(e_charge, wpa_ref[...], preferred_element_type=jnp.float32)
         + jnp.dot(e_temp, wpb_ref[...], preferred_element_type=jnp.float32))
    out_ref[...] = y * jax.nn.sigmoid(y)


def _make_gather(n, d, n_g):
    rows_per_w = n_g * _G
    mesh = plsc.VectorSubcoreMesh(core_axis_name="c", subcore_axis_name="s")

    @functools.partial(
        pl.kernel,
        mesh=mesh,
        out_type=jax.ShapeDtypeStruct((n, d), jnp.float32),
        scratch_types=[
            pltpu.VMEM((rows_per_w,), jnp.int32),
        ] + [pltpu.VMEM((_G, d), jnp.float32) for _ in range(_NBUF)]
          + [pltpu.SemaphoreType.DMA for _ in range(2 * _NBUF)],
        compiler_params=pltpu.CompilerParams(use_tc_tiling_on_sc=False),
    )
    def gather_k(table_hbm, batch_hbm, out_hbm, idx_v, *bufs):
        rows = bufs[:_NBUF]
        gs = bufs[_NBUF:2 * _NBUF]
        ss = bufs[2 * _NBUF:]
        wid = lax.axis_index("s") * _NC + lax.axis_index("c")
        woff = jnp.minimum(wid * rows_per_w, n - rows_per_w)
        pltpu.sync_copy(batch_hbm.at[pl.ds(woff, rows_per_w)], idx_v)

        def fire_gather(j, p):
            pltpu.async_copy(table_hbm.at[idx_v.at[pl.ds(j * _G, _G)]],
                             rows[p], gs[p])

        def wait_gather(j, p):
            pltpu.make_async_copy(table_hbm.at[idx_v.at[pl.ds(j * _G, _G)]],
                                  rows[p], gs[p]).wait()

        def fire_store(j, p):
            pltpu.async_copy(rows[p], out_hbm.at[pl.ds(woff + j * _G, _G)],
                             ss[p])

        def wait_store(j, p):
            pltpu.make_async_copy(rows[p],
                                  out_hbm.at[pl.ds(woff + j * _G, _G)],
                                  ss[p]).wait()

        for j in range(_FA):
            fire_gather(j, j)

        def ring(jj, carry):
            for bq in range(_NBUF):
                j = jj * _NBUF + bq
                q = (bq + _FA) % _NBUF

                @pl.when(j >= _NBUF - _FA)
                def _():
                    wait_store(j - (_NBUF - _FA), q)

                @pl.when(j + _FA < n_g)
                def _():
                    fire_gather(j + _FA, q)

                wait_gather(j, bq)
                fire_store(j, bq)
            return carry

        lax.fori_loop(0, n_g // _NBUF, ring, 0, unroll=False)
        for j in range(n_g - (_NBUF - _FA), n_g):
            wait_store(j, j % _NBUF)

    return gather_k


def kernel(batch, charge, temperature, emb_charge, W1, W2, Wp):
    n = batch.shape[0]
    b = charge.shape[0]
    v, d = emb_charge.shape

    vp = ((v + 127) // 128) * 128
    emb_pad = jnp.zeros((vp, d), jnp.float32).at[:v].set(emb_charge)
    table = pl.pallas_call(
        _table_body,
        out_shape=jax.ShapeDtypeStruct((b, d), jnp.float32),
    )(charge.astype(jnp.int32).reshape(b, 1),
      temperature.reshape(b, 1),
      emb_pad, W1, W2, Wp[:d, :], Wp[d:, :])

    chunk = _NW * _G
    n_g = -(-n // chunk)
    n_g = ((n_g + _NBUF - 1) // _NBUF) * _NBUF
    assert n_g * _G <= n, "worker window exceeds output"
    return _make_gather(n, d, n_g)(table, batch.astype(jnp.int32))


# speedup vs baseline: 6.6939x; 1.1122x over previous
import functools

import jax
import jax.numpy as jnp
from jax import lax
from jax.experimental import pallas as pl
from jax.experimental.pallas import tpu as pltpu
from jax.experimental.pallas import tpu_sc as plsc

_NC = 2
_NS = 16
_NW = _NC * _NS
_CH = 448
_NSL = 7
_L = 16


def _table_body(charge_ref, temp_ref, emb_ref, w1_ref, w2_ref, wpa_ref,
                wpb_ref, out_ref):
    b = charge_ref.shape[0]
    vp = emb_ref.shape[0]
    charge = charge_ref[...]
    iota = lax.broadcasted_iota(jnp.int32, (b, vp), 1)
    oh = jnp.where(charge == iota, 1.0, 0.0).astype(jnp.float32)
    e_charge = jnp.dot(oh, emb_ref[...],
                       preferred_element_type=jnp.float32)
    t = temp_ref[...]
    h = t * w1_ref[...]
    h = h * jax.nn.sigmoid(h)
    e_temp = jnp.dot(h, w2_ref[...],
                     preferred_element_type=jnp.float32)
    y = (jnp.dot(e_charge, wpa_ref[...], preferred_element_type=jnp.float32)
         + jnp.dot(e_temp, wpb_ref[...], preferred_element_type=jnp.float32))
    out_ref[...] = y * jax.nn.sigmoid(y)


def _make_expand(n, b, d):
    atoms_per_w = _NSL * _CH
    mesh = plsc.VectorSubcoreMesh(core_axis_name="c", subcore_axis_name="s")

    @functools.partial(
        pl.kernel,
        mesh=mesh,
        out_type=jax.ShapeDtypeStruct((d, n), jnp.float32),
        scratch_types=[
            pltpu.VMEM((b, d), jnp.float32),
            pltpu.VMEM((atoms_per_w,), jnp.int32),
            pltpu.VMEM((d, _CH), jnp.float32),
            pltpu.VMEM((d, _CH), jnp.float32),
            pltpu.SemaphoreType.DMA,
            pltpu.SemaphoreType.DMA,
        ],
        compiler_params=pltpu.CompilerParams(use_tc_tiling_on_sc=False,
                                             needs_layout_passes=False),
    )
    def expand_k(table_hbm, batch_hbm, out_hbm, table_v, idx_v,
                 sl0, sl1, ss0, ss1):
        slabs = (sl0, sl1)
        ss = (ss0, ss1)
        wid = lax.axis_index("s") * _NC + lax.axis_index("c")
        woff = jnp.minimum(wid * atoms_per_w, n - atoms_per_w)
        pltpu.sync_copy(table_hbm, table_v)
        pltpu.sync_copy(batch_hbm.at[pl.ds(woff, atoms_per_w)], idx_v)

        def store(sl, p):
            pltpu.async_copy(
                slabs[p], out_hbm.at[:, pl.ds(woff + sl * _CH, _CH)], ss[p])

        def wait_store(sl, p):
            pltpu.make_async_copy(
                slabs[p], out_hbm.at[:, pl.ds(woff + sl * _CH, _CH)],
                ss[p]).wait()

        for sl in range(_NSL):
            p = sl % 2
            if sl >= 2:
                wait_store(sl - 2, p)
            slab = slabs[p]

            def group(g, carry):
                idx16 = idx_v[pl.ds(sl * _CH + g * _L, _L)]
                for f in range(d):
                    fv = jnp.full((_L,), f, jnp.int32)
                    v = plsc.load_gather(table_v, [idx16, fv])
                    slab[f, pl.ds(g * _L, _L)] = v
                return carry

            lax.fori_loop(0, _CH // _L, group, 0, unroll=False)
            store(sl, p)
        wait_store(_NSL - 2, _NSL % 2)
        wait_store(_NSL - 1, (_NSL - 1) % 2)

    return expand_k


def kernel(batch, charge, temperature, emb_charge, W1, W2, Wp):
    n = batch.shape[0]
    b = charge.shape[0]
    v, d = emb_charge.shape

    vp = ((v + 127) // 128) * 128
    emb_pad = jnp.zeros((vp, d), jnp.float32).at[:v].set(emb_charge)
    table = pl.pallas_call(
        _table_body,
        out_shape=jax.ShapeDtypeStruct((b, d), jnp.float32),
    )(charge.astype(jnp.int32).reshape(b, 1),
      temperature.reshape(b, 1),
      emb_pad, W1, W2, Wp[:d, :], Wp[d:, :])

    assert _NW * _NSL * _CH >= n and _NSL * _CH <= n
    out_t = _make_expand(n, b, d)(table, batch.astype(jnp.int32))
    return out_t.T


# speedup vs baseline: 9.0612x; 1.3537x over previous
import functools

import jax
import jax.numpy as jnp
from jax import lax
from jax.experimental import pallas as pl
from jax.experimental.pallas import tpu as pltpu
from jax.experimental.pallas import tpu_sc as plsc

_NC = 2
_NS = 16
_NW = _NC * _NS
_CH = 448
_NSL = 7
_L = 16


def _table_body(charge_ref, temp_ref, emb_ref, w1_ref, w2_ref, wpa_ref,
                wpb_ref, out_ref):
    b = charge_ref.shape[0]
    vp = emb_ref.shape[0]
    charge = charge_ref[...]
    iota = lax.broadcasted_iota(jnp.int32, (b, vp), 1)
    oh = jnp.where(charge == iota, 1.0, 0.0).astype(jnp.float32)
    e_charge = jnp.dot(oh, emb_ref[...],
                       preferred_element_type=jnp.float32)
    t = temp_ref[...]
    h = t * w1_ref[...]
    h = h * jax.nn.sigmoid(h)
    e_temp = jnp.dot(h, w2_ref[...],
                     preferred_element_type=jnp.float32)
    y = (jnp.dot(e_charge, wpa_ref[...], preferred_element_type=jnp.float32)
         + jnp.dot(e_temp, wpb_ref[...], preferred_element_type=jnp.float32))
    out_ref[...] = y * jax.nn.sigmoid(y)


def _make_expand(n, b, d):
    atoms_per_w = _NSL * _CH
    mesh = plsc.VectorSubcoreMesh(core_axis_name="c", subcore_axis_name="s")

    @functools.partial(
        pl.kernel,
        mesh=mesh,
        out_type=jax.ShapeDtypeStruct((d, n), jnp.float32),
        scratch_types=[
            pltpu.VMEM((b, d), jnp.float32),
            pltpu.VMEM((atoms_per_w,), jnp.int32),
            pltpu.VMEM((d, _CH), jnp.float32),
            pltpu.VMEM((d, _CH), jnp.float32),
            pltpu.SemaphoreType.DMA,
            pltpu.SemaphoreType.DMA,
        ],
        compiler_params=pltpu.CompilerParams(use_tc_tiling_on_sc=False,
                                             needs_layout_passes=False),
    )
    def expand_k(table_hbm, batch_hbm, out_hbm, table_v, idx_v,
                 sl0, sl1, ss0, ss1):
        slabs = (sl0, sl1)
        ss = (ss0, ss1)
        wid = lax.axis_index("s") * _NC + lax.axis_index("c")
        woff = jnp.minimum(wid * atoms_per_w, n - atoms_per_w)
        pltpu.sync_copy(table_hbm, table_v)
        pltpu.sync_copy(batch_hbm.at[pl.ds(woff, atoms_per_w)], idx_v)

        def store(sl, p):
            pltpu.async_copy(
                slabs[p], out_hbm.at[:, pl.ds(woff + sl * _CH, _CH)], ss[p])

        def wait_store(sl, p):
            pltpu.make_async_copy(
                slabs[p], out_hbm.at[:, pl.ds(woff + sl * _CH, _CH)],
                ss[p]).wait()

        for sl in range(_NSL):
            p = sl % 2
            if sl >= 2:
                wait_store(sl - 2, p)
            slab = slabs[p]

            @plsc.parallel_loop(0, _CH, step=_L, unroll=1)
            def _group(i):
                idx16 = idx_v[pl.ds(sl * _CH + i, _L)]
                for f in range(d):
                    fv = jnp.full((_L,), f, jnp.int32)
                    v = plsc.load_gather(table_v, [idx16, fv])
                    slab[f, pl.ds(i, _L)] = v
            store(sl, p)
        wait_store(_NSL - 2, _NSL % 2)
        wait_store(_NSL - 1, (_NSL - 1) % 2)

    return expand_k


def kernel(batch, charge, temperature, emb_charge, W1, W2, Wp):
    n = batch.shape[0]
    b = charge.shape[0]
    v, d = emb_charge.shape

    vp = ((v + 127) // 128) * 128
    emb_pad = jnp.zeros((vp, d), jnp.float32).at[:v].set(emb_charge)
    table = pl.pallas_call(
        _table_body,
        out_shape=jax.ShapeDtypeStruct((b, d), jnp.float32),
    )(charge.astype(jnp.int32).reshape(b, 1),
      temperature.reshape(b, 1),
      emb_pad, W1, W2, Wp[:d, :], Wp[d:, :])

    assert _NW * _NSL * _CH >= n and _NSL * _CH <= n
    out_t = _make_expand(n, b, d)(table, batch.astype(jnp.int32))
    return out_t.T
